# baseline (speedup 1.0000x reference)
import functools

import jax
import jax.numpy as jnp
from jax import lax
from jax.experimental import pallas as pl
from jax.experimental.pallas import tpu as pltpu
from jax.experimental.pallas import tpu_sc as plsc

_NUM_CORES = 2
_NUM_SUBCORES = 16
_NUM_WORKERS = _NUM_CORES * _NUM_SUBCORES


def _pad_body(t_ref, o_ref):
    x = t_ref[...]
    o_ref[...] = jnp.concatenate([x, jnp.zeros_like(x)], axis=1)


def _pad_table(table, r_tile):
    v, d = table.shape
    return pl.pallas_call(
        _pad_body,
        grid=(pl.cdiv(v, r_tile),),
        in_specs=[pl.BlockSpec((r_tile, d), lambda i: (i, 0))],
        out_specs=pl.BlockSpec((r_tile, 2 * d), lambda i: (i, 0)),
        out_shape=jax.ShapeDtypeStruct((v, 2 * d), jnp.float32),
        compiler_params=pltpu.CompilerParams(
            dimension_semantics=("parallel",),
        ),
    )(table)


def _sc_gather(idx, table2):
    (n,) = idx.shape
    _, d2 = table2.shape
    per_w = n // _NUM_WORKERS
    mesh = plsc.VectorSubcoreMesh(core_axis_name="c", subcore_axis_name="s")

    @functools.partial(
        pl.kernel,
        mesh=mesh,
        out_type=jax.ShapeDtypeStruct((n, d2), jnp.float32),
        compiler_params=pltpu.CompilerParams(use_tc_tiling_on_sc=True),
        scratch_types=[
            pltpu.VMEM((per_w,), jnp.int32),
            pltpu.VMEM((per_w, d2), jnp.float32),
            pltpu.SemaphoreType.DMA,
        ],
    )
    def gather_kernel(idx_hbm, table_hbm, out_hbm, idx_v, rows_v, sem):
        wid = lax.axis_index("s") * _NUM_CORES + lax.axis_index("c")
        base = wid * per_w
        pltpu.sync_copy(idx_hbm.at[pl.ds(base, per_w)], idx_v)
        pltpu.async_copy(table_hbm.at[idx_v], rows_v, sem).wait()
        pltpu.sync_copy(rows_v, out_hbm.at[pl.ds(base, per_w)])

    return gather_kernel(idx, table2)


def _matmul_body(x2_ref, pos_ref, w_ref, b_ref, out_ref, xp_ref):
    @pl.when(pl.program_id(0) == 0)
    def _():
        d = xp_ref.shape[1]
        xp_ref[...] = x2_ref[:, :d] + pos_ref[...]

    out_ref[0] = (
        jnp.dot(xp_ref[...], w_ref[...], preferred_element_type=jnp.float32)
        + b_ref[...]
    )


def _lm_head(x2, pos, w, b2, v_tile):
    t, d = pos.shape
    v = w.shape[1]
    nvt = pl.cdiv(v, v_tile)
    return pl.pallas_call(
        _matmul_body,
        grid=(nvt,),
        in_specs=[
            pl.BlockSpec((t, 2 * d), lambda j: (0, 0)),
            pl.BlockSpec((t, d), lambda j: (0, 0)),
            pl.BlockSpec((d, v_tile), lambda j: (0, j)),
            pl.BlockSpec((1, v_tile), lambda j: (0, j)),
        ],
        out_specs=pl.BlockSpec((1, t, v_tile), lambda j: (0, 0, j)),
        out_shape=jax.ShapeDtypeStruct((1, t, v), jnp.float32),
        scratch_shapes=[pltpu.VMEM((t, d), jnp.float32)],
        compiler_params=pltpu.CompilerParams(
            dimension_semantics=("arbitrary",),
        ),
    )(x2, pos, w, b2)


def kernel(indices, token_table, pos_table, W, b):
    batch, seq = indices.shape
    idx = indices.reshape(-1).astype(jnp.int32)
    table2 = _pad_table(token_table, r_tile=8192)
    x2 = _sc_gather(idx, table2)
    return _lm_head(x2, pos_table[:seq], W, b.reshape(1, -1), v_tile=1024)


# speedup vs baseline: 3.0522x; 3.0522x over previous
import functools

import jax
import jax.numpy as jnp
from jax import lax
from jax.experimental import pallas as pl
from jax.experimental.pallas import tpu as pltpu
from jax.experimental.pallas import tpu_sc as plsc

_NUM_CORES = 2
_NUM_SUBCORES = 16
_NUM_WORKERS = _NUM_CORES * _NUM_SUBCORES


def _pad_body(t_ref, o_ref):
    x = t_ref[...]
    o_ref[...] = jnp.concatenate([x, jnp.zeros_like(x)], axis=1)


def _pad_table(table, r_tile):
    v, d = table.shape
    return pl.pallas_call(
        _pad_body,
        grid=(pl.cdiv(v, r_tile),),
        in_specs=[pl.BlockSpec((r_tile, d), lambda i: (i, 0))],
        out_specs=pl.BlockSpec((r_tile, 2 * d), lambda i: (i, 0)),
        out_shape=jax.ShapeDtypeStruct((v, 2 * d), jnp.float32),
        compiler_params=pltpu.CompilerParams(
            dimension_semantics=("parallel",),
        ),
    )(table)


def _sc_gather(idx, table2):
    (n,) = idx.shape
    _, d2 = table2.shape
    per_w = n // _NUM_WORKERS
    mesh = plsc.VectorSubcoreMesh(core_axis_name="c", subcore_axis_name="s")

    @functools.partial(
        pl.kernel,
        mesh=mesh,
        out_type=jax.ShapeDtypeStruct((n, d2), jnp.float32),
        compiler_params=pltpu.CompilerParams(use_tc_tiling_on_sc=True),
        scratch_types=[
            pltpu.VMEM((per_w,), jnp.int32),
            pltpu.VMEM((per_w, d2), jnp.float32),
            pltpu.SemaphoreType.DMA,
        ],
    )
    def gather_kernel(idx_hbm, table_hbm, out_hbm, idx_v, rows_v, sem):
        wid = lax.axis_index("s") * _NUM_CORES + lax.axis_index("c")
        base = wid * per_w
        pltpu.sync_copy(idx_hbm.at[pl.ds(base, per_w)], idx_v)
        pltpu.async_copy(table_hbm.at[idx_v], rows_v, sem).wait()
        pltpu.sync_copy(rows_v, out_hbm.at[pl.ds(base, per_w)])

    return gather_kernel(idx, table2)


def _matmul_body(x2_ref, pos_ref, w_ref, b_ref, out_ref, xpt_ref):
    @pl.when(pl.program_id(0) == 0)
    def _():
        d = pos_ref.shape[1]
        xp = x2_ref[:, :d] + pos_ref[...]
        xpt_ref[...] = xp.T

    acc = jax.lax.dot_general(
        w_ref[...],
        xpt_ref[...],
        (((0,), (0,)), ((), ())),
        preferred_element_type=jnp.float32,
    )
    out_ref[...] = acc + b_ref[...].T


def _lm_head(x2, pos, w, b2, v_tile):
    t, d = pos.shape
    v = w.shape[1]
    nvt = pl.cdiv(v, v_tile)
    return pl.pallas_call(
        _matmul_body,
        grid=(nvt,),
        in_specs=[
            pl.BlockSpec((t, 2 * d), lambda j: (0, 0)),
            pl.BlockSpec((t, d), lambda j: (0, 0)),
            pl.BlockSpec((d, v_tile), lambda j: (0, j)),
            pl.BlockSpec((1, v_tile), lambda j: (0, j)),
        ],
        out_specs=pl.BlockSpec((v_tile, t), lambda j: (j, 0)),
        out_shape=jax.ShapeDtypeStruct((v, t), jnp.float32),
        scratch_shapes=[pltpu.VMEM((d, t), jnp.float32)],
        compiler_params=pltpu.CompilerParams(
            dimension_semantics=("arbitrary",),
            fuse_transposed_lhs_in_matmul=True,
        ),
    )(x2, pos, w, b2)


def kernel(indices, token_table, pos_table, W, b):
    batch, seq = indices.shape
    idx = indices.reshape(-1).astype(jnp.int32)
    table2 = _pad_table(token_table, r_tile=8192)
    x2 = _sc_gather(idx, table2)
    logits_t = _lm_head(x2, pos_table[:seq], W, b.reshape(1, -1), v_tile=1024)
    return logits_t.T[None]


# speedup vs baseline: 3.2600x; 1.0681x over previous
import functools

import jax
import jax.numpy as jnp
from jax import lax
from jax.experimental import pallas as pl
from jax.experimental.pallas import tpu as pltpu
from jax.experimental.pallas import tpu_sc as plsc

_NUM_CORES = 2
_NUM_SUBCORES = 16
_NUM_WORKERS = _NUM_CORES * _NUM_SUBCORES


def _pad_body(tt_ref, o_ref):
    xt = tt_ref[...].T
    o_ref[...] = jnp.concatenate([xt, jnp.zeros_like(xt)], axis=1)


def _pad_table(table_t, r_tile):
    d, v = table_t.shape
    return pl.pallas_call(
        _pad_body,
        grid=(pl.cdiv(v, r_tile),),
        in_specs=[pl.BlockSpec((d, r_tile), lambda i: (0, i))],
        out_specs=pl.BlockSpec((r_tile, 2 * d), lambda i: (i, 0)),
        out_shape=jax.ShapeDtypeStruct((v, 2 * d), jnp.float32),
        compiler_params=pltpu.CompilerParams(
            dimension_semantics=("parallel",),
        ),
    )(table_t)


def _sc_gather(idx, table2):
    (n,) = idx.shape
    _, d2 = table2.shape
    per_w = n // _NUM_WORKERS
    mesh = plsc.VectorSubcoreMesh(core_axis_name="c", subcore_axis_name="s")

    @functools.partial(
        pl.kernel,
        mesh=mesh,
        out_type=jax.ShapeDtypeStruct((n, d2), jnp.float32),
        compiler_params=pltpu.CompilerParams(use_tc_tiling_on_sc=True),
        scratch_types=[
            pltpu.VMEM((per_w,), jnp.int32),
            pltpu.VMEM((per_w, d2), jnp.float32),
            pltpu.SemaphoreType.DMA,
        ],
    )
    def gather_kernel(idx_hbm, table_hbm, out_hbm, idx_v, rows_v, sem):
        wid = lax.axis_index("s") * _NUM_CORES + lax.axis_index("c")
        base = wid * per_w
        pltpu.sync_copy(idx_hbm.at[pl.ds(base, per_w)], idx_v)
        pltpu.async_copy(table_hbm.at[idx_v], rows_v, sem).wait()
        pltpu.sync_copy(rows_v, out_hbm.at[pl.ds(base, per_w)])

    return gather_kernel(idx, table2)


def _matmul_body(x2_ref, pos_ref, w_ref, b_ref, out_ref, xpt_ref):
    @pl.when(pl.program_id(0) == 0)
    def _():
        d = pos_ref.shape[1]
        xp = x2_ref[:, :d] + pos_ref[...]
        xpt_ref[...] = xp.T

    acc = jax.lax.dot_general(
        w_ref[...],
        xpt_ref[...],
        (((0,), (0,)), ((), ())),
        preferred_element_type=jnp.float32,
    )
    out_ref[...] = acc + b_ref[...].T


def _lm_head(x2, pos, w, b2, v_tile):
    t, d = pos.shape
    v = w.shape[1]
    nvt = pl.cdiv(v, v_tile)
    return pl.pallas_call(
        _matmul_body,
        grid=(nvt,),
        in_specs=[
            pl.BlockSpec((t, 2 * d), lambda j: (0, 0)),
            pl.BlockSpec((t, d), lambda j: (0, 0)),
            pl.BlockSpec((d, v_tile), lambda j: (0, j)),
            pl.BlockSpec((1, v_tile), lambda j: (0, j)),
        ],
        out_specs=pl.BlockSpec((v_tile, t), lambda j: (j, 0)),
        out_shape=jax.ShapeDtypeStruct((v, t), jnp.float32),
        scratch_shapes=[pltpu.VMEM((d, t), jnp.float32)],
        compiler_params=pltpu.CompilerParams(
            dimension_semantics=("arbitrary",),
            fuse_transposed_lhs_in_matmul=True,
        ),
    )(x2, pos, w, b2)


def kernel(indices, token_table, pos_table, W, b):
    batch, seq = indices.shape
    idx = indices.reshape(-1).astype(jnp.int32)
    table2 = _pad_table(token_table.T, r_tile=2048)
    x2 = _sc_gather(idx, table2)
    logits_t = _lm_head(x2, pos_table[:seq], W, b.reshape(1, -1), v_tile=1024)
    return logits_t.T[None]


# speedup vs baseline: 3.3079x; 1.0147x over previous
import functools

import jax
import jax.numpy as jnp
from jax import lax
from jax.experimental import pallas as pl
from jax.experimental.pallas import tpu as pltpu
from jax.experimental.pallas import tpu_sc as plsc

_NUM_CORES = 2
_NUM_SUBCORES = 16
_NUM_WORKERS = _NUM_CORES * _NUM_SUBCORES
_LANES = 16
_FIRE = 16


def _sc_gather_t(idx, flat_t, n_dims, vocab):
    (n,) = idx.shape
    d_grp = 8
    n_tok_grp = _NUM_WORKERS // (n_dims // d_grp)
    per_w = n // n_tok_grp
    mesh = plsc.VectorSubcoreMesh(core_axis_name="c", subcore_axis_name="s")

    @functools.partial(
        pl.kernel,
        mesh=mesh,
        out_type=jax.ShapeDtypeStruct((n_dims * n,), jnp.float32),
        compiler_params=pltpu.CompilerParams(use_tc_tiling_on_sc=False),
        scratch_types=[
            pltpu.VMEM((per_w,), jnp.int32),
            pltpu.VMEM((d_grp * per_w,), jnp.int32),
            pltpu.VMEM((d_grp * per_w,), jnp.float32),
            pltpu.SemaphoreType.DMA,
        ],
    )
    def gather_kernel(idx_hbm, tab_hbm, out_hbm, idx_v, idxs_v, xt_v, sem):
        wid = lax.axis_index("s") * _NUM_CORES + lax.axis_index("c")
        g = wid // n_tok_grp
        q = wid % n_tok_grp
        base = q * per_w
        pltpu.sync_copy(idx_hbm.at[pl.ds(base, per_w)], idx_v)
        for k in range(per_w // _LANES):
            sl = pl.ds(k * _LANES, _LANES)
            v = idx_v[sl] + g * (d_grp * vocab)
            for d in range(d_grp):
                idxs_v[pl.ds(d * per_w + k * _LANES, _LANES)] = v + d * vocab
        copies = [
            pltpu.async_copy(
                tab_hbm.at[idxs_v.at[pl.ds(d * per_w, per_w)]],
                xt_v.at[pl.ds(d * per_w, per_w)],
                sem,
            )
            for d in range(d_grp)
        ]
        for c in copies:
            c.wait()
        for d in range(d_grp):
            pltpu.sync_copy(
                xt_v.at[pl.ds(d * per_w, per_w)],
                out_hbm.at[pl.ds((g * d_grp + d) * n + base, per_w)],
            )

    return gather_kernel(idx, flat_t)


def _matmul_body(xt_ref, post_ref, w_ref, b_ref, out_ref, xpt_ref):
    @pl.when(pl.program_id(0) == 0)
    def _():
        xpt_ref[...] = xt_ref[...] + post_ref[...]

    acc = jax.lax.dot_general(
        w_ref[...],
        xpt_ref[...],
        (((0,), (0,)), ((), ())),
        preferred_element_type=jnp.float32,
    )
    out_ref[...] = acc + b_ref[...].T


def _lm_head(xt, post, w, b2, v_tile):
    d, t = post.shape
    v = w.shape[1]
    nvt = pl.cdiv(v, v_tile)
    return pl.pallas_call(
        _matmul_body,
        grid=(nvt,),
        in_specs=[
            pl.BlockSpec((d, t), lambda j: (0, 0)),
            pl.BlockSpec((d, t), lambda j: (0, 0)),
            pl.BlockSpec((d, v_tile), lambda j: (0, j)),
            pl.BlockSpec((1, v_tile), lambda j: (0, j)),
        ],
        out_specs=pl.BlockSpec((v_tile, t), lambda j: (j, 0)),
        out_shape=jax.ShapeDtypeStruct((v, t), jnp.float32),
        scratch_shapes=[pltpu.VMEM((d, t), jnp.float32)],
        compiler_params=pltpu.CompilerParams(
            dimension_semantics=("arbitrary",),
            fuse_transposed_lhs_in_matmul=True,
        ),
    )(xt, post, w, b2)


def kernel(indices, token_table, pos_table, W, b):
    batch, seq = indices.shape
    vocab, dim = token_table.shape
    idx = indices.reshape(-1).astype(jnp.int32)
    flat_t = token_table.T.reshape(-1)
    xt = _sc_gather_t(idx, flat_t, dim, vocab).reshape(dim, batch * seq)
    logits_t = _lm_head(
        xt, pos_table[:seq].T, W, b.reshape(1, -1), v_tile=1024
    )
    return logits_t.T[None]
